# Initial kernel scaffold; baseline (speedup 1.0000x reference)
import jax
import jax.numpy as jnp
from jax.experimental import pallas as pl


def kernel(ast_paths_node_indices, ast_paths_lengths, ast_paths_mask, ast_nodes_types, ident_leaf_identifier_idx, ident_leaf_nodes_indices, prim_leaf_types, prim_leaf_nodes_indices, mod_leaf_mods, mod_leaf_nodes_indices, identifiers_encodings, ast_paths_child_place, ast_paths_vertical_direction, node_type_emb, prim_emb, mod_emb, orient_emb, W_ident, b_ident, W_prim, b_prim, W_mod, b_mod, W_orient, b_orient, W_seq, b_seq):
    raise NotImplementedError("write your pallas kernel here")


# speedup vs baseline: 4.7386x; 4.7386x over previous
import functools

import jax
import jax.numpy as jnp
from jax import lax
from jax.experimental import pallas as pl

M = 200000
P = 8192
L = 24
D = 128
K = 20000
NB = 1000

F32 = jnp.float32


def _prep_body(emb_ref, Ws_ref, bs_ref, Wi_ref, Wp_ref, Wm_ref,
               pemb_ref, memb_ref, bi_ref, bp_ref, bm_ref,
               T_ref, C3_ref, Zpm_ref, WiZ_ref, b3_ref):
    emb = emb_ref[...]
    Ws = Ws_ref[...]
    bs = bs_ref[...]
    dot = functools.partial(jnp.dot, preferred_element_type=F32)
    T_ref[...] = jnp.zeros((256, D), F32)
    T_ref[0:200, :] = jnp.maximum(dot(emb, Ws) + bs, 0.0)
    C3_ref[...] = jnp.zeros((3, 256, D), F32)
    C3_ref[0, 0:200, :] = dot(emb, dot(Wi_ref[0:D, :], Ws))
    C3_ref[1, 0:200, :] = dot(emb, dot(Wp_ref[0:D, :], Ws))
    C3_ref[2, 0:200, :] = dot(emb, dot(Wm_ref[0:D, :], Ws))
    WiZ_ref[...] = dot(Wi_ref[D:, :], Ws)
    Zpm_ref[0:16, :] = dot(pemb_ref[...], dot(Wp_ref[D:, :], Ws))
    Zpm_ref[16:32, :] = dot(memb_ref[...], dot(Wm_ref[D:, :], Ws))
    b3_ref[0:1, :] = dot(bi_ref[...], Ws) + bs
    b3_ref[1:2, :] = dot(bp_ref[...], Ws) + bs
    b3_ref[2:3, :] = dot(bm_ref[...], Ws) + bs


def _prep(emb, Ws, bs, Wi, Wp, Wm, pemb, memb, bi, bp, bm):
    return pl.pallas_call(
        _prep_body,
        out_shape=(
            jax.ShapeDtypeStruct((256, D), F32),
            jax.ShapeDtypeStruct((3, 256, D), F32),
            jax.ShapeDtypeStruct((32, D), F32),
            jax.ShapeDtypeStruct((D, D), F32),
            jax.ShapeDtypeStruct((3, D), F32),
        ),
    )(emb, Ws, bs.reshape(1, D), Wi, Wp, Wm, pemb, memb,
      bi.reshape(1, D), bp.reshape(1, D), bm.reshape(1, D))


def _rbuild_body(t_ref, G_ref, pm_ref, C3_ref, Zpm_ref, WiZ_ref, b3_ref,
                 R_ref):
    p = pl.program_id(0)
    dot = functools.partial(jnp.dot, preferred_element_type=F32)
    t = t_ref[...]
    oh = (lax.broadcasted_iota(jnp.int32, (NB, 256), 1) == t).astype(F32)
    base = dot(oh, C3_ref[0])
    gi = dot(G_ref[...], WiZ_ref[...])
    pm = pm_ref[...] + jnp.where(p >= 40, 16, 0)
    oh2 = (lax.broadcasted_iota(jnp.int32, (NB, 32), 1) == pm).astype(F32)
    gpm = dot(oh2, Zpm_ref[...])
    part = jnp.where(p < 20, gi, gpm)
    R_ref[...] = jnp.maximum(base + part + b3_ref[0], 0.0)


def _rbuild(t_all, G, pm, C3, Zpm, WiZ, b3):
    nblk = (3 * K) // NB
    gblk = K // NB
    return pl.pallas_call(
        _rbuild_body,
        grid=(nblk,),
        in_specs=[
            pl.BlockSpec((NB, 1), lambda p: (p, 0)),
            pl.BlockSpec((NB, D), lambda p: (jnp.minimum(p, gblk - 1), 0)),
            pl.BlockSpec((NB, 1),
                         lambda p: (jnp.clip(p - gblk, 0, 2 * gblk - 1), 0)),
            pl.BlockSpec((1, 256, D), lambda p: (p // gblk, 0, 0)),
            pl.BlockSpec((32, D), lambda p: (0, 0)),
            pl.BlockSpec((D, D), lambda p: (0, 0)),
            pl.BlockSpec((1, 1, D), lambda p: (p // gblk, 0, 0)),
        ],
        out_specs=pl.BlockSpec((NB, D), lambda p: (p, 0)),
        out_shape=jax.ShapeDtypeStruct((3 * K, D), F32),
    )(t_all.reshape(3 * K, 1), G, pm.reshape(2 * K, 1),
      C3, Zpm, WiZ, b3.reshape(3, 1, D))


def _blend_body(t_ref, c0_ref, c1_ref, w_ref, patch_ref, T_ref, out_ref):
    dot = functools.partial(jnp.dot, preferred_element_type=F32)
    t = t_ref[...]
    oh = (lax.broadcasted_iota(jnp.int32, (NB, 256), 1) == t).astype(F32)
    base = dot(oh, T_ref[...])
    cnt = c0_ref[...] + c1_ref[...]
    sel = jnp.where(w_ref[...] > 0, patch_ref[...], base)
    out_ref[...] = cnt * sel


def _blend(types, c0, c1, win, patch, T):
    nblk = M // NB
    return pl.pallas_call(
        _blend_body,
        grid=(nblk,),
        in_specs=[
            pl.BlockSpec((NB, 1), lambda p: (p, 0)),
            pl.BlockSpec((NB, 1), lambda p: (p, 0)),
            pl.BlockSpec((NB, 1), lambda p: (p, 0)),
            pl.BlockSpec((NB, 1), lambda p: (p, 0)),
            pl.BlockSpec((NB, D), lambda p: (p, 0)),
            pl.BlockSpec((256, D), lambda p: (0, 0)),
        ],
        out_specs=pl.BlockSpec((NB, D), lambda p: (p, 0)),
        out_shape=jax.ShapeDtypeStruct((M, D), F32),
    )(types.reshape(M, 1), c0.reshape(M, 1), c1.reshape(M, 1),
      win.reshape(M, 1), patch, T)


def kernel(ast_paths_node_indices, ast_paths_lengths, ast_paths_mask,
           ast_nodes_types, ident_leaf_identifier_idx, ident_leaf_nodes_indices,
           prim_leaf_types, prim_leaf_nodes_indices, mod_leaf_mods,
           mod_leaf_nodes_indices, identifiers_encodings, ast_paths_child_place,
           ast_paths_vertical_direction, node_type_emb, prim_emb, mod_emb,
           orient_emb, W_ident, b_ident, W_prim, b_prim, W_mod, b_mod,
           W_orient, b_orient, W_seq, b_seq):
    types = ast_nodes_types.astype(jnp.int32)
    pidx = ast_paths_node_indices.astype(jnp.int32).reshape(-1)
    all_idx = jnp.concatenate([
        ident_leaf_nodes_indices, prim_leaf_nodes_indices,
        mod_leaf_nodes_indices]).astype(jnp.int32)
    pm = jnp.concatenate([prim_leaf_types, mod_leaf_mods]).astype(jnp.int32)

    T, C3, Zpm, WiZ, b3 = _prep(node_type_emb, W_seq, b_seq, W_ident, W_prim,
                                W_mod, prim_emb, mod_emb, b_ident, b_prim,
                                b_mod)

    counts = jnp.zeros((M,), F32).at[pidx].add(1.0)
    c0, c1 = counts, jnp.zeros((M,), F32)
    t_all = types[all_idx]
    G = identifiers_encodings[ident_leaf_identifier_idx.astype(jnp.int32)]
    win = jnp.zeros((M,), jnp.int32).at[all_idx].max(
        jnp.arange(3 * K, dtype=jnp.int32) + 1)

    R = _rbuild(t_all, G, pm, C3, Zpm, WiZ, b3)

    patch = R[jnp.maximum(win - 1, 0)]

    return _blend(types, c0, c1, win, patch, T)


# speedup vs baseline: 5.0848x; 1.0731x over previous
import functools

import jax
import jax.numpy as jnp
from jax import lax
from jax.experimental import pallas as pl
from jax.experimental.pallas import tpu as pltpu
from jax.experimental.pallas import tpu_sc as plsc

M = 200000
P = 8192
L = 24
D = 128
K = 20000
NB = 1000

F32 = jnp.float32
I32 = jnp.int32

NC = 2
NS = 16
NW = NC * NS
MP = 200704
OWN = MP // NW
CSL = MP // NS
NPIDX = P * L
APAD = 61440
IPAD = 20480
NUPD = 3 * K


def _prep_body(emb_ref, Ws_ref, bs_ref, Wi_ref, Wp_ref, Wm_ref,
               pemb_ref, memb_ref, bi_ref, bp_ref, bm_ref,
               T_ref, C3_ref, Zpm_ref, WiZ_ref, b3_ref):
    emb = emb_ref[...]
    Ws = Ws_ref[...]
    bs = bs_ref[...]
    dot = functools.partial(jnp.dot, preferred_element_type=F32)
    T_ref[...] = jnp.zeros((256, D), F32)
    T_ref[0:200, :] = jnp.maximum(dot(emb, Ws) + bs, 0.0)
    C3_ref[...] = jnp.zeros((3, 256, D), F32)
    C3_ref[0, 0:200, :] = dot(emb, dot(Wi_ref[0:D, :], Ws))
    C3_ref[1, 0:200, :] = dot(emb, dot(Wp_ref[0:D, :], Ws))
    C3_ref[2, 0:200, :] = dot(emb, dot(Wm_ref[0:D, :], Ws))
    WiZ_ref[...] = dot(Wi_ref[D:, :], Ws)
    Zpm_ref[0:16, :] = dot(pemb_ref[...], dot(Wp_ref[D:, :], Ws))
    Zpm_ref[16:32, :] = dot(memb_ref[...], dot(Wm_ref[D:, :], Ws))
    b3_ref[0:1, :] = dot(bi_ref[...], Ws) + bs
    b3_ref[1:2, :] = dot(bp_ref[...], Ws) + bs
    b3_ref[2:3, :] = dot(bm_ref[...], Ws) + bs


def _prep(emb, Ws, bs, Wi, Wp, Wm, pemb, memb, bi, bp, bm):
    return pl.pallas_call(
        _prep_body,
        out_shape=(
            jax.ShapeDtypeStruct((256, D), F32),
            jax.ShapeDtypeStruct((3, 256, D), F32),
            jax.ShapeDtypeStruct((32, D), F32),
            jax.ShapeDtypeStruct((D, D), F32),
            jax.ShapeDtypeStruct((3, D), F32),
        ),
    )(emb, Ws, bs.reshape(1, D), Wi, Wp, Wm, pemb, memb,
      bi.reshape(1, D), bp.reshape(1, D), bm.reshape(1, D))


def _rbuild_body(t_ref, G_ref, pm_ref, C3_ref, Zpm_ref, WiZ_ref, b3_ref,
                 R_ref):
    p = pl.program_id(0)
    dot = functools.partial(jnp.dot, preferred_element_type=F32)
    t = t_ref[...]
    oh = (lax.broadcasted_iota(jnp.int32, (NB, 256), 1) == t).astype(F32)
    base = dot(oh, C3_ref[0])
    gi = dot(G_ref[...], WiZ_ref[...])
    pm = pm_ref[...] + jnp.where(p >= 40, 16, 0)
    oh2 = (lax.broadcasted_iota(jnp.int32, (NB, 32), 1) == pm).astype(F32)
    gpm = dot(oh2, Zpm_ref[...])
    part = jnp.where(p < 20, gi, gpm)
    R_ref[...] = jnp.maximum(base + part + b3_ref[0], 0.0)


def _rbuild(t_all, G, pm, C3, Zpm, WiZ, b3):
    nblk = (3 * K) // NB
    gblk = K // NB
    return pl.pallas_call(
        _rbuild_body,
        grid=(nblk,),
        in_specs=[
            pl.BlockSpec((NB, 1), lambda p: (p, 0)),
            pl.BlockSpec((NB, D), lambda p: (jnp.minimum(p, gblk - 1), 0)),
            pl.BlockSpec((NB, 1),
                         lambda p: (jnp.clip(p - gblk, 0, 2 * gblk - 1), 0)),
            pl.BlockSpec((1, 256, D), lambda p: (p // gblk, 0, 0)),
            pl.BlockSpec((32, D), lambda p: (0, 0)),
            pl.BlockSpec((D, D), lambda p: (0, 0)),
            pl.BlockSpec((1, 1, D), lambda p: (p // gblk, 0, 0)),
        ],
        out_specs=pl.BlockSpec((NB, D), lambda p: (p, 0)),
        out_shape=jax.ShapeDtypeStruct((3 * K, D), F32),
    )(t_all.reshape(3 * K, 1), G, pm.reshape(2 * K, 1),
      C3, Zpm, WiZ, b3.reshape(3, 1, D))


def _blend_body(t_ref, c0_ref, c1_ref, w_ref, patch_ref, T_ref, out_ref):
    dot = functools.partial(jnp.dot, preferred_element_type=F32)
    t = t_ref[...]
    oh = (lax.broadcasted_iota(jnp.int32, (NB, 256), 1) == t).astype(F32)
    base = dot(oh, T_ref[...])
    cnt = c0_ref[...] + c1_ref[...]
    sel = jnp.where(w_ref[...] > 0, patch_ref[...], base)
    out_ref[...] = cnt * sel


def _blend(types, c0, c1, win, patch, T):
    nblk = M // NB
    return pl.pallas_call(
        _blend_body,
        grid=(nblk,),
        in_specs=[
            pl.BlockSpec((NB, 1), lambda p: (p, 0)),
            pl.BlockSpec((NB, 1), lambda p: (p, 0)),
            pl.BlockSpec((NB, 1), lambda p: (p, 0)),
            pl.BlockSpec((NB, 1), lambda p: (p, 0)),
            pl.BlockSpec((NB, D), lambda p: (p, 0)),
            pl.BlockSpec((256, D), lambda p: (0, 0)),
        ],
        out_specs=pl.BlockSpec((NB, D), lambda p: (p, 0)),
        out_shape=jax.ShapeDtypeStruct((M, D), F32),
    )(types.reshape(M, 1), c0.reshape(M, 1), c1.reshape(M, 1),
      win.reshape(M, 1), patch, T)


def _iota16():
    return lax.iota(I32, 16)


def _popcount_via(pcb, mask):
    del pcb
    vals = jnp.where(mask, jnp.ones((16,), I32), jnp.zeros((16,), I32))
    s = vals[0]
    for l in range(1, 16):
        s = s + vals[l]
    return s


def _sc1_body(pidx2d, aidx1d, ids1d, types_h, identenc,
              counts_o, tall_o, G_out,
              sh_counts, idxb, onesb, zb, aib, tb, iib, gb,
              semb, semc, semd):
    c = lax.axis_index("c")
    s = lax.axis_index("s")
    wid = s * NC + c
    it16 = _iota16()

    def zero_zb(i, _):
        zb[pl.ds(i * 16, 16)] = jnp.zeros((16,), F32)
        return 0
    lax.fori_loop(0, CSL // 16, zero_zb, 0)
    pltpu.sync_copy(zb, sh_counts.at[pl.ds(s * CSL, CSL)])

    plsc.subcore_barrier()

    pltpu.sync_copy(pidx2d.at[pl.ds(wid * 48, 48)], idxb)
    for j in range(8):
        onesb[0, pl.ds(j * 16, 16)] = jnp.ones((16,), F32)
    descs = []
    for j in range(48):
        descs.append(pltpu.async_copy(
            onesb.at[0], sh_counts.at[idxb.at[j]], semb, add=True))
    pltpu.sync_copy(aidx1d.at[pl.ds(wid * 1920, 1920)], aib)
    tdescs = []
    for j in range(15):
        tdescs.append(pltpu.async_copy(
            types_h.at[aib.at[pl.ds(j * 128, 128)]],
            tb.at[pl.ds(j * 128, 128)], semc))
    for dsc in tdescs:
        dsc.wait()
    pltpu.sync_copy(tb, tall_o.at[pl.ds(wid * 1920, 1920)])
    pltpu.sync_copy(ids1d.at[pl.ds(wid * 640, 640)], iib)
    for j in range(5):
        pltpu.async_copy(identenc.at[iib.at[pl.ds(j * 128, 128)]], gb,
                         semd).wait()
        pltpu.sync_copy(gb, G_out.at[pl.ds((wid * 5 + j) * 128, 128)])
    for dsc in descs:
        dsc.wait()
    plsc.subcore_barrier()

    pltpu.sync_copy(sh_counts.at[pl.ds(s * CSL, CSL)], zb)
    pltpu.sync_copy(zb, counts_o.at[pl.ds(c * MP + s * CSL, CSL)])

def _sc1(pidx2d, aidx1d, ids1d, types_h, identenc):
    mesh = plsc.VectorSubcoreMesh(core_axis_name="c", subcore_axis_name="s",
                                  num_cores=NC, num_subcores=NS)
    f = pl.kernel(
        _sc1_body,
        out_type=(
            jax.ShapeDtypeStruct((NC * MP,), F32),
            jax.ShapeDtypeStruct((APAD,), I32),
            jax.ShapeDtypeStruct((IPAD, D), F32),
        ),
        mesh=mesh,
        scratch_types=[
            pltpu.VMEM_SHARED((MP,), F32),
            pltpu.VMEM((48, 128), I32),
            pltpu.VMEM((1, 128), F32),
            pltpu.VMEM((CSL,), F32),
            pltpu.VMEM((1920,), I32),
            pltpu.VMEM((1920,), I32),
            pltpu.VMEM((640,), I32),
            pltpu.VMEM((128, D), F32),
            pltpu.SemaphoreType.DMA,
            pltpu.SemaphoreType.DMA,
            pltpu.SemaphoreType.DMA,
        ],
    )
    return f(pidx2d, aidx1d, ids1d, types_h, identenc)


def _sc2_body(nodes_l, winj_l, cnt16, R_h, patch, cbuf, jbuf, idx2d, rowb,
              sem):
    c = lax.axis_index("c")
    s = lax.axis_index("s")
    wid = s * NC + c

    pltpu.sync_copy(cnt16, cbuf)
    cnt = cbuf[pl.ds(0, 16)][0]
    nch = (cnt + 127) // 128

    def chunk_step(k, _):
        ch = wid + k * NW

        @pl.when(ch < nch)
        def _():
            pltpu.sync_copy(winj_l.at[pl.ds(ch * 128, 128)], jbuf)
            pltpu.async_copy(R_h.at[jbuf], rowb, sem).wait()
            pltpu.sync_copy(nodes_l.at[pl.ds(ch * 128, 128)], idx2d.at[0])
            pltpu.sync_copy(rowb, patch.at[idx2d.at[0]])
        return 0
    lax.fori_loop(0, (NUPD // 128 + NW - 1) // NW, chunk_step, 0)


def _sc2(nodes_l, winj_l, cnt16, R_h):
    mesh = plsc.VectorSubcoreMesh(core_axis_name="c", subcore_axis_name="s",
                                  num_cores=NC, num_subcores=NS)
    f = pl.kernel(
        _sc2_body,
        out_type=jax.ShapeDtypeStruct((MP, D), F32),
        mesh=mesh,
        scratch_types=[
            pltpu.VMEM((16,), I32),
            pltpu.VMEM((128,), I32),
            pltpu.VMEM((1, 128), I32),
            pltpu.VMEM((128, D), F32),
            pltpu.SemaphoreType.DMA,
        ],
    )
    return f(nodes_l, winj_l, cnt16, R_h)


def kernel(ast_paths_node_indices, ast_paths_lengths, ast_paths_mask,
           ast_nodes_types, ident_leaf_identifier_idx, ident_leaf_nodes_indices,
           prim_leaf_types, prim_leaf_nodes_indices, mod_leaf_mods,
           mod_leaf_nodes_indices, identifiers_encodings, ast_paths_child_place,
           ast_paths_vertical_direction, node_type_emb, prim_emb, mod_emb,
           orient_emb, W_ident, b_ident, W_prim, b_prim, W_mod, b_mod,
           W_orient, b_orient, W_seq, b_seq):
    types = ast_nodes_types.astype(jnp.int32)
    pidx = ast_paths_node_indices.astype(jnp.int32).reshape(-1)
    all_idx = jnp.concatenate([
        ident_leaf_nodes_indices, prim_leaf_nodes_indices,
        mod_leaf_nodes_indices]).astype(jnp.int32)
    pm = jnp.concatenate([prim_leaf_types, mod_leaf_mods]).astype(jnp.int32)

    T, C3, Zpm, WiZ, b3 = _prep(node_type_emb, W_seq, b_seq, W_ident, W_prim,
                                W_mod, prim_emb, mod_emb, b_ident, b_prim,
                                b_mod)

    pidx2d = pidx.reshape(NPIDX // 128, 128)
    aidx1d = jnp.concatenate([all_idx, jnp.zeros((APAD - NUPD,), I32)])
    ids1d = jnp.concatenate(
        [ident_leaf_identifier_idx.astype(I32),
         jnp.zeros((IPAD - K,), I32)])

    counts_o, tall_o, G = _sc1(pidx2d, aidx1d, ids1d, types,
                               identifiers_encodings)
    c0 = counts_o[:M]
    c1 = counts_o[MP:MP + M]
    t_all = tall_o[:NUPD]

    jpos = jnp.arange(NUPD, dtype=I32)
    win_pad = jnp.zeros((MP,), I32).at[all_idx].max(jpos + 1, mode="drop")
    is_win = win_pad[all_idx] == jpos + 1
    cpos = jnp.cumsum(is_win.astype(I32))
    dest = jnp.where(is_win, cpos - 1, NUPD)
    nodes_l = jnp.full((NUPD + 128,), MP - 1, I32).at[dest].set(all_idx)
    winj_l = jnp.zeros((NUPD + 128,), I32).at[dest].set(jpos)
    cnt16 = jnp.full((16,), cpos[-1], I32)

    R = _rbuild(t_all, G, pm, C3, Zpm, WiZ, b3)

    patch = _sc2(nodes_l, winj_l, cnt16, R)

    return _blend(types, c0, c1, win_pad[:M], patch, T)


# speedup vs baseline: 6.0121x; 1.1824x over previous
import functools

import jax
import jax.numpy as jnp
from jax import lax
from jax.experimental import pallas as pl
from jax.experimental.pallas import tpu as pltpu
from jax.experimental.pallas import tpu_sc as plsc

M = 200000
P = 8192
L = 24
D = 128
K = 20000
NB = 1000

F32 = jnp.float32
I32 = jnp.int32

NC = 2
NS = 16
NW = NC * NS
MP = 200704
OWN = MP // NW
CSL = MP // NS
NPIDX = P * L
APAD = 61440
IPAD = 20480
NUPD = 3 * K


def _prep_body(emb_ref, Ws_ref, bs_ref, Wi_ref, Wp_ref, Wm_ref,
               pemb_ref, memb_ref, bi_ref, bp_ref, bm_ref,
               T_ref, C3_ref, Zpm_ref, WiZ_ref, b3_ref):
    emb = emb_ref[...]
    Ws = Ws_ref[...]
    bs = bs_ref[...]
    dot = functools.partial(jnp.dot, preferred_element_type=F32)
    T_ref[...] = jnp.zeros((256, D), F32)
    T_ref[0:200, :] = jnp.maximum(dot(emb, Ws) + bs, 0.0)
    C3_ref[...] = jnp.zeros((3, 256, D), F32)
    C3_ref[0, 0:200, :] = dot(emb, dot(Wi_ref[0:D, :], Ws))
    C3_ref[1, 0:200, :] = dot(emb, dot(Wp_ref[0:D, :], Ws))
    C3_ref[2, 0:200, :] = dot(emb, dot(Wm_ref[0:D, :], Ws))
    WiZ_ref[...] = dot(Wi_ref[D:, :], Ws)
    Zpm_ref[0:16, :] = dot(pemb_ref[...], dot(Wp_ref[D:, :], Ws))
    Zpm_ref[16:32, :] = dot(memb_ref[...], dot(Wm_ref[D:, :], Ws))
    b3_ref[0:1, :] = dot(bi_ref[...], Ws) + bs
    b3_ref[1:2, :] = dot(bp_ref[...], Ws) + bs
    b3_ref[2:3, :] = dot(bm_ref[...], Ws) + bs


def _prep(emb, Ws, bs, Wi, Wp, Wm, pemb, memb, bi, bp, bm):
    return pl.pallas_call(
        _prep_body,
        out_shape=(
            jax.ShapeDtypeStruct((256, D), F32),
            jax.ShapeDtypeStruct((3, 256, D), F32),
            jax.ShapeDtypeStruct((32, D), F32),
            jax.ShapeDtypeStruct((D, D), F32),
            jax.ShapeDtypeStruct((3, D), F32),
        ),
    )(emb, Ws, bs.reshape(1, D), Wi, Wp, Wm, pemb, memb,
      bi.reshape(1, D), bp.reshape(1, D), bm.reshape(1, D))


def _rbuild_body(t_ref, G_ref, pm_ref, C3_ref, Zpm_ref, WiZ_ref, b3_ref,
                 R_ref):
    p = pl.program_id(0)
    dot = functools.partial(jnp.dot, preferred_element_type=F32)
    t = t_ref[...]
    oh = (lax.broadcasted_iota(jnp.int32, (NB, 256), 1) == t).astype(F32)
    base = dot(oh, C3_ref[0])
    gi = dot(G_ref[...], WiZ_ref[...])
    pm = pm_ref[...] + jnp.where(p >= 40, 16, 0)
    oh2 = (lax.broadcasted_iota(jnp.int32, (NB, 32), 1) == pm).astype(F32)
    gpm = dot(oh2, Zpm_ref[...])
    part = jnp.where(p < 20, gi, gpm)
    R_ref[...] = jnp.maximum(base + part + b3_ref[0], 0.0)


def _rbuild(t_all, G, pm, C3, Zpm, WiZ, b3):
    nblk = (3 * K) // NB
    gblk = K // NB
    return pl.pallas_call(
        _rbuild_body,
        grid=(nblk,),
        in_specs=[
            pl.BlockSpec((NB, 1), lambda p: (p, 0)),
            pl.BlockSpec((NB, D), lambda p: (jnp.minimum(p, gblk - 1), 0)),
            pl.BlockSpec((NB, 1),
                         lambda p: (jnp.clip(p - gblk, 0, 2 * gblk - 1), 0)),
            pl.BlockSpec((1, 256, D), lambda p: (p // gblk, 0, 0)),
            pl.BlockSpec((32, D), lambda p: (0, 0)),
            pl.BlockSpec((D, D), lambda p: (0, 0)),
            pl.BlockSpec((1, 1, D), lambda p: (p // gblk, 0, 0)),
        ],
        out_specs=pl.BlockSpec((NB, D), lambda p: (p, 0)),
        out_shape=jax.ShapeDtypeStruct((3 * K, D), F32),
    )(t_all.reshape(3 * K, 1), G, pm.reshape(2 * K, 1),
      C3, Zpm, WiZ, b3.reshape(3, 1, D))


def _blend_body(t_ref, c0_ref, c1_ref, u0_ref, u1_ref, patch_ref, T_ref,
                out_ref):
    dot = functools.partial(jnp.dot, preferred_element_type=F32)
    t = t_ref[...]
    oh = (lax.broadcasted_iota(jnp.int32, (NB, 256), 1) == t).astype(F32)
    base = dot(oh, T_ref[...])
    cnt = c0_ref[...] + c1_ref[...]
    upd = u0_ref[...] + u1_ref[...]
    sel = jnp.where(upd > 0, patch_ref[...], base)
    out_ref[...] = cnt * sel


def _blend(types, c0, c1, u0, u1, patch, T):
    nblk = M // NB
    return pl.pallas_call(
        _blend_body,
        grid=(nblk,),
        in_specs=[
            pl.BlockSpec((NB, 1), lambda p: (p, 0)),
            pl.BlockSpec((NB, 1), lambda p: (p, 0)),
            pl.BlockSpec((NB, 1), lambda p: (p, 0)),
            pl.BlockSpec((NB, 1), lambda p: (p, 0)),
            pl.BlockSpec((NB, 1), lambda p: (p, 0)),
            pl.BlockSpec((NB, D), lambda p: (p, 0)),
            pl.BlockSpec((256, D), lambda p: (0, 0)),
        ],
        out_specs=pl.BlockSpec((NB, D), lambda p: (p, 0)),
        out_shape=jax.ShapeDtypeStruct((M, D), F32),
    )(types.reshape(M, 1), c0.reshape(M, 1), c1.reshape(M, 1),
      u0.reshape(M, 1), u1.reshape(M, 1), patch, T)


def _iota16():
    return lax.iota(I32, 16)


def _popcount_via(pcb, mask):
    del pcb
    vals = jnp.where(mask, jnp.ones((16,), I32), jnp.zeros((16,), I32))
    s = vals[0]
    for l in range(1, 16):
        s = s + vals[l]
    return s


def _sc1_body(pidx2d, aidx1d, ids1d, types_h, identenc,
              counts_o, updc_o, tall_o, G_out,
              sh_counts, sh_upd, idxb, onesb, zb, aib, aib2, tb, iib, gb,
              semb, semc, semd):
    c = lax.axis_index("c")
    s = lax.axis_index("s")
    wid = s * NC + c
    it16 = _iota16()

    def zero_zb(i, _):
        zb[pl.ds(i * 16, 16)] = jnp.zeros((16,), F32)
        return 0
    lax.fori_loop(0, CSL // 16, zero_zb, 0)
    pltpu.sync_copy(zb, sh_counts.at[pl.ds(s * CSL, CSL)])
    pltpu.sync_copy(zb, sh_upd.at[pl.ds(s * CSL, CSL)])

    plsc.subcore_barrier()

    pltpu.sync_copy(pidx2d.at[pl.ds(wid * 48, 48)], idxb)
    for j in range(8):
        onesb[0, pl.ds(j * 16, 16)] = jnp.ones((16,), F32)
    descs = []
    for j in range(48):
        descs.append(pltpu.async_copy(
            onesb.at[0], sh_counts.at[idxb.at[j]], semb, add=True))
    for j in range(15):
        pltpu.sync_copy(aidx1d.at[pl.ds(wid * 1920 + j * 128, 128)],
                        aib2.at[j])
    for j in range(15):
        descs.append(pltpu.async_copy(
            onesb.at[0], sh_upd.at[aib2.at[j]], semb, add=True))
    pltpu.sync_copy(aidx1d.at[pl.ds(wid * 1920, 1920)], aib)
    tdescs = []
    for j in range(15):
        tdescs.append(pltpu.async_copy(
            types_h.at[aib.at[pl.ds(j * 128, 128)]],
            tb.at[pl.ds(j * 128, 128)], semc))
    for dsc in tdescs:
        dsc.wait()
    pltpu.sync_copy(tb, tall_o.at[pl.ds(wid * 1920, 1920)])
    pltpu.sync_copy(ids1d.at[pl.ds(wid * 640, 640)], iib)
    for j in range(5):
        pltpu.async_copy(identenc.at[iib.at[pl.ds(j * 128, 128)]], gb,
                         semd).wait()
        pltpu.sync_copy(gb, G_out.at[pl.ds((wid * 5 + j) * 128, 128)])
    for dsc in descs:
        dsc.wait()
    plsc.subcore_barrier()

    pltpu.sync_copy(sh_counts.at[pl.ds(s * CSL, CSL)], zb)
    pltpu.sync_copy(zb, counts_o.at[pl.ds(c * MP + s * CSL, CSL)])
    pltpu.sync_copy(sh_upd.at[pl.ds(s * CSL, CSL)], zb)
    pltpu.sync_copy(zb, updc_o.at[pl.ds(c * MP + s * CSL, CSL)])

def _sc1(pidx2d, aidx1d, ids1d, types_h, identenc):
    mesh = plsc.VectorSubcoreMesh(core_axis_name="c", subcore_axis_name="s",
                                  num_cores=NC, num_subcores=NS)
    f = pl.kernel(
        _sc1_body,
        out_type=(
            jax.ShapeDtypeStruct((NC * MP,), F32),
            jax.ShapeDtypeStruct((NC * MP,), F32),
            jax.ShapeDtypeStruct((APAD,), I32),
            jax.ShapeDtypeStruct((IPAD, D), F32),
        ),
        mesh=mesh,
        scratch_types=[
            pltpu.VMEM_SHARED((MP,), F32),
            pltpu.VMEM_SHARED((MP,), F32),
            pltpu.VMEM((48, 128), I32),
            pltpu.VMEM((1, 128), F32),
            pltpu.VMEM((CSL,), F32),
            pltpu.VMEM((1920,), I32),
            pltpu.VMEM((15, 128), I32),
            pltpu.VMEM((1920,), I32),
            pltpu.VMEM((640,), I32),
            pltpu.VMEM((128, D), F32),
            pltpu.SemaphoreType.DMA,
            pltpu.SemaphoreType.DMA,
            pltpu.SemaphoreType.DMA,
        ],
    )
    return f(pidx2d, aidx1d, ids1d, types_h, identenc)


def _sc2_body(nodes_l, winj_l, R_h, patch, jbuf, idx2d, rowb, sem):
    c = lax.axis_index("c")
    s = lax.axis_index("s")
    wid = s * NC + c
    nch = 60032 // 128

    def chunk_step(k, _):
        ch = wid + k * NW

        @pl.when(ch < nch)
        def _():
            pltpu.sync_copy(winj_l.at[pl.ds(ch * 128, 128)], jbuf)
            pltpu.async_copy(R_h.at[jbuf], rowb, sem).wait()
            pltpu.sync_copy(nodes_l.at[pl.ds(ch * 128, 128)], idx2d.at[0])
            pltpu.sync_copy(rowb, patch.at[idx2d.at[0]])
        return 0
    lax.fori_loop(0, (nch + NW - 1) // NW, chunk_step, 0)


def _sc2(nodes_l, winj_l, R_h):
    mesh = plsc.VectorSubcoreMesh(core_axis_name="c", subcore_axis_name="s",
                                  num_cores=NC, num_subcores=NS)
    f = pl.kernel(
        _sc2_body,
        out_type=jax.ShapeDtypeStruct((MP, D), F32),
        mesh=mesh,
        scratch_types=[
            pltpu.VMEM((128,), I32),
            pltpu.VMEM((1, 128), I32),
            pltpu.VMEM((128, D), F32),
            pltpu.SemaphoreType.DMA,
        ],
    )
    return f(nodes_l, winj_l, R_h)


def kernel(ast_paths_node_indices, ast_paths_lengths, ast_paths_mask,
           ast_nodes_types, ident_leaf_identifier_idx, ident_leaf_nodes_indices,
           prim_leaf_types, prim_leaf_nodes_indices, mod_leaf_mods,
           mod_leaf_nodes_indices, identifiers_encodings, ast_paths_child_place,
           ast_paths_vertical_direction, node_type_emb, prim_emb, mod_emb,
           orient_emb, W_ident, b_ident, W_prim, b_prim, W_mod, b_mod,
           W_orient, b_orient, W_seq, b_seq):
    types = ast_nodes_types.astype(jnp.int32)
    pidx = ast_paths_node_indices.astype(jnp.int32).reshape(-1)
    all_idx = jnp.concatenate([
        ident_leaf_nodes_indices, prim_leaf_nodes_indices,
        mod_leaf_nodes_indices]).astype(jnp.int32)
    pm = jnp.concatenate([prim_leaf_types, mod_leaf_mods]).astype(jnp.int32)

    T, C3, Zpm, WiZ, b3 = _prep(node_type_emb, W_seq, b_seq, W_ident, W_prim,
                                W_mod, prim_emb, mod_emb, b_ident, b_prim,
                                b_mod)

    pidx2d = pidx.reshape(NPIDX // 128, 128)
    aidx1d = jnp.concatenate([all_idx, jnp.zeros((APAD - NUPD,), I32)])
    ids1d = jnp.concatenate(
        [ident_leaf_identifier_idx.astype(I32),
         jnp.zeros((IPAD - K,), I32)])

    counts_o, updc_o, tall_o, G = _sc1(pidx2d, aidx1d, ids1d, types,
                                       identifiers_encodings)
    c0 = counts_o[:M]
    c1 = counts_o[MP:MP + M]
    u0 = updc_o[:M]
    u1 = updc_o[MP:MP + M]
    t_all = tall_o[:NUPD]

    jpos = jnp.arange(NUPD, dtype=I32)
    win_pad = jnp.zeros((MP,), I32).at[all_idx].max(jpos + 1, mode="drop")
    is_win = win_pad[all_idx] == jpos + 1
    nodes_l = jnp.concatenate(
        [jnp.where(is_win, all_idx, MP - 1), jnp.full((32,), MP - 1, I32)])
    winj_l = jnp.concatenate([jpos, jnp.zeros((32,), I32)])

    R = _rbuild(t_all, G, pm, C3, Zpm, WiZ, b3)

    patch = _sc2(nodes_l, winj_l, R)

    return _blend(types, c0, c1, u0, u1, patch, T)
